# Initial kernel scaffold; baseline (speedup 1.0000x reference)
import jax
import jax.numpy as jnp
from jax.experimental import pallas as pl


def kernel(x):
    raise NotImplementedError("write your pallas kernel here")


# speedup vs baseline: 33.5681x; 33.5681x over previous
import functools

import jax
import jax.numpy as jnp
import numpy as np
from jax.experimental import pallas as pl

N = 8192
N1 = 64
N2 = 128
KH = 40
F_MAX = N // 2
TOPK = 4
DT = 128


def _dft_consts():
    kl = np.arange(N2)[:, None]
    n2 = np.arange(N2)[None, :]
    ph = -2.0 * np.pi * ((kl * n2) % N2) / N2
    d128r = np.cos(ph).astype(np.float32)
    d128i = np.sin(ph).astype(np.float32)

    n1 = np.arange(N1)[None, :]
    pht = -2.0 * np.pi * ((np.arange(N2)[:, None] * n1) % N) / N
    tr = np.cos(pht).astype(np.float32)
    ti = np.sin(pht).astype(np.float32)

    kh = np.arange(KH)[:, None]
    phe = -2.0 * np.pi * ((kh * n1) % N1) / N1
    er = np.cos(phe).astype(np.float32)
    ei = np.sin(phe).astype(np.float32)
    return d128r, d128i, tr, ti, er, ei


_D128R, _D128I, _TR, _TI, _ER, _EI = _dft_consts()


def _fa_kernel(x_ref, d128r_ref, d128i_ref, tr_ref, ti_ref, er_ref, ei_ref,
               out_ref):
    xb = x_ref[0]
    a2f = xb.reshape(N2, N1 * DT)
    hi_p = jax.lax.Precision.HIGHEST
    gr = jnp.dot(d128r_ref[...], a2f, precision=hi_p)
    gi = jnp.dot(d128i_ref[...], a2f, precision=hi_p)
    gr3 = gr.reshape(N2, N1, DT)
    gi3 = gi.reshape(N2, N1, DT)
    tr3 = tr_ref[...][:, :, None]
    ti3 = ti_ref[...][:, :, None]
    hr = gr3 * tr3 - gi3 * ti3
    hi = gr3 * ti3 + gi3 * tr3
    hpr = jnp.transpose(hr, (1, 0, 2)).reshape(N1, N2 * DT)
    hpi = jnp.transpose(hi, (1, 0, 2)).reshape(N1, N2 * DT)
    er_m = er_ref[...]
    ei_m = ei_ref[...]
    x2r = (jnp.dot(er_m, hpr, precision=hi_p)
           - jnp.dot(ei_m, hpi, precision=hi_p)).reshape(KH * N2, DT)
    x2i = (jnp.dot(er_m, hpi, precision=hi_p)
           + jnp.dot(ei_m, hpr, precision=hi_p)).reshape(KH * N2, DT)
    nrow = KH * N2
    kidx = jax.lax.broadcasted_iota(jnp.int32, (nrow, DT), 0)
    amp2 = x2r * x2r + x2i * x2i
    amp2 = jnp.where(kidx <= F_MAX, amp2, -1.0)

    out3 = jnp.zeros((N2, N1, DT), jnp.float32)
    a_io = jax.lax.broadcasted_iota(jnp.int32, (N1, DT), 0)
    b_io = jax.lax.broadcasted_iota(jnp.int32, (N2, DT), 0)
    work = amp2
    inv_n = np.float32(1.0 / N)
    for _ in range(TOPK):
        m = jnp.max(work, axis=0, keepdims=True)
        cand = jnp.where(work == m, kidx, jnp.int32(2**30))
        f_sel = jnp.min(cand, axis=0, keepdims=True)
        onehot = kidx == f_sel
        wr = jnp.sum(jnp.where(onehot, x2r, 0.0), axis=0, keepdims=True)
        wi = jnp.sum(jnp.where(onehot, x2i, 0.0), axis=0, keepdims=True)
        work = jnp.where(onehot, -2.0, work)
        scale = jnp.where((f_sel == 0) | (f_sel == F_MAX), inv_n,
                          np.float32(2.0 / N))
        cr = wr * scale
        ci = wi * scale
        fa = (a_io * f_sel) & (N - 1)
        th_a = fa.astype(jnp.float32) * np.float32(2.0 * np.pi / N)
        uc = jnp.cos(th_a)
        us = jnp.sin(th_a)
        fb = (b_io * f_sel) & (N2 - 1)
        th_b = fb.astype(jnp.float32) * np.float32(2.0 * np.pi / N2)
        vc = jnp.cos(th_b)
        vs = jnp.sin(th_b)
        p = cr * uc - ci * us
        q = -(cr * us + ci * uc)
        out3 = out3 + vc[:, None, :] * p[None, :, :] + vs[:, None, :] * q[None, :, :]

    out_ref[0] = out3.reshape(N, DT)


@jax.jit
def kernel(x):
    b, n, d = x.shape
    grid = (b, d // DT)
    const_spec = lambda shp: pl.BlockSpec(shp, lambda i, j: (0, 0))
    return pl.pallas_call(
        _fa_kernel,
        grid=grid,
        in_specs=[
            pl.BlockSpec((1, N, DT), lambda i, j: (i, 0, j)),
            const_spec((N2, N2)), const_spec((N2, N2)),
            const_spec((N2, N1)), const_spec((N2, N1)),
            const_spec((KH, N1)), const_spec((KH, N1)),
        ],
        out_specs=pl.BlockSpec((1, N, DT), lambda i, j: (i, 0, j)),
        out_shape=jax.ShapeDtypeStruct((b, n, d), jnp.float32),
    )(x, _D128R, _D128I, _TR, _TI, _ER, _EI)


# speedup vs baseline: 49.7753x; 1.4828x over previous
import functools

import jax
import jax.numpy as jnp
import numpy as np
from jax.experimental import pallas as pl

N = 8192
N1 = 64
N2 = 128
KH = 40
F_MAX = N // 2
TOPK = 4
DT = 128


def _dft_consts():
    kl = np.arange(N2)[:, None]
    n2 = np.arange(N2)[None, :]
    ph = -2.0 * np.pi * ((kl * n2) % N2) / N2
    d128r = np.cos(ph).astype(np.float32)
    d128i = np.sin(ph).astype(np.float32)

    n1 = np.arange(N1)[None, :]
    pht = -2.0 * np.pi * ((np.arange(N2)[:, None] * n1) % N) / N
    tr = np.cos(pht).astype(np.float32)
    ti = np.sin(pht).astype(np.float32)

    kh = np.arange(KH)[:, None]
    phe = -2.0 * np.pi * ((kh * n1) % N1) / N1
    er = np.cos(phe).astype(np.float32)
    ei = np.sin(phe).astype(np.float32)
    return d128r, d128i, tr, ti, er, ei


_D128R, _D128I, _TR, _TI, _ER, _EI = _dft_consts()


def _fa_kernel(x_ref, d128r_ref, d128i_ref, tr_ref, ti_ref, er_ref, ei_ref,
               out_ref):
    xb = x_ref[0]
    a2f = xb.reshape(N2, N1 * DT)
    hi_p = jax.lax.Precision.DEFAULT
    gr = jnp.dot(d128r_ref[...], a2f, precision=hi_p)
    gi = jnp.dot(d128i_ref[...], a2f, precision=hi_p)
    gr3 = gr.reshape(N2, N1, DT)
    gi3 = gi.reshape(N2, N1, DT)
    tr3 = tr_ref[...][:, :, None]
    ti3 = ti_ref[...][:, :, None]
    hr = gr3 * tr3 - gi3 * ti3
    hi = gr3 * ti3 + gi3 * tr3
    hpr = jnp.transpose(hr, (1, 0, 2)).reshape(N1, N2 * DT)
    hpi = jnp.transpose(hi, (1, 0, 2)).reshape(N1, N2 * DT)
    er_m = er_ref[...]
    ei_m = ei_ref[...]
    x2r = (jnp.dot(er_m, hpr, precision=hi_p)
           - jnp.dot(ei_m, hpi, precision=hi_p)).reshape(KH * N2, DT)
    x2i = (jnp.dot(er_m, hpi, precision=hi_p)
           + jnp.dot(ei_m, hpr, precision=hi_p)).reshape(KH * N2, DT)
    nrow = KH * N2
    kidx = jax.lax.broadcasted_iota(jnp.int32, (nrow, DT), 0)
    amp2 = x2r * x2r + x2i * x2i
    amp2 = jnp.where(kidx <= F_MAX, amp2, -1.0)

    out3 = jnp.zeros((N2, N1, DT), jnp.float32)
    a_io = jax.lax.broadcasted_iota(jnp.int32, (N1, DT), 0)
    b_io = jax.lax.broadcasted_iota(jnp.int32, (N2, DT), 0)
    work = amp2
    inv_n = np.float32(1.0 / N)
    for _ in range(TOPK):
        m = jnp.max(work, axis=0, keepdims=True)
        cand = jnp.where(work == m, kidx, jnp.int32(2**30))
        f_sel = jnp.min(cand, axis=0, keepdims=True)
        onehot = kidx == f_sel
        wr = jnp.sum(jnp.where(onehot, x2r, 0.0), axis=0, keepdims=True)
        wi = jnp.sum(jnp.where(onehot, x2i, 0.0), axis=0, keepdims=True)
        work = jnp.where(onehot, -2.0, work)
        scale = jnp.where((f_sel == 0) | (f_sel == F_MAX), inv_n,
                          np.float32(2.0 / N))
        cr = wr * scale
        ci = wi * scale
        fa = (a_io * f_sel) & (N - 1)
        th_a = fa.astype(jnp.float32) * np.float32(2.0 * np.pi / N)
        uc = jnp.cos(th_a)
        us = jnp.sin(th_a)
        fb = (b_io * f_sel) & (N2 - 1)
        th_b = fb.astype(jnp.float32) * np.float32(2.0 * np.pi / N2)
        vc = jnp.cos(th_b)
        vs = jnp.sin(th_b)
        p = cr * uc - ci * us
        q = -(cr * us + ci * uc)
        out3 = out3 + vc[:, None, :] * p[None, :, :] + vs[:, None, :] * q[None, :, :]

    out_ref[0] = out3.reshape(N, DT)


@jax.jit
def kernel(x):
    b, n, d = x.shape
    grid = (b, d // DT)
    const_spec = lambda shp: pl.BlockSpec(shp, lambda i, j: (0, 0))
    return pl.pallas_call(
        _fa_kernel,
        grid=grid,
        in_specs=[
            pl.BlockSpec((1, N, DT), lambda i, j: (i, 0, j)),
            const_spec((N2, N2)), const_spec((N2, N2)),
            const_spec((N2, N1)), const_spec((N2, N1)),
            const_spec((KH, N1)), const_spec((KH, N1)),
        ],
        out_specs=pl.BlockSpec((1, N, DT), lambda i, j: (i, 0, j)),
        out_shape=jax.ShapeDtypeStruct((b, n, d), jnp.float32),
    )(x, _D128R, _D128I, _TR, _TI, _ER, _EI)
